# baseline (speedup 1.0000x reference)
import functools

import jax
import jax.numpy as jnp
from jax import lax
from jax.experimental import pallas as pl
from jax.experimental.pallas import tpu as pltpu
from jax.experimental.pallas import tpu_sc as plsc

DIM = 256
LANES = 16
K32 = DIM // 32
HDIM = DIM // 2
NC, NS = 2, 16
NW = NC * NS
CHUNK = 128


def _tc_body(z_ref, ws_ref, wd_ref, bias_ref, a_ref, b_ref):
    zb = z_ref[...]
    a_ref[...] = (
        jnp.dot(zb, ws_ref[...], preferred_element_type=jnp.float32)
        + bias_ref[...]
    )
    b_ref[...] = jnp.dot(zb, wd_ref[...], preferred_element_type=jnp.float32)


def _node_transform(z, W_src, W_dst, bias2d):
    n, d = z.shape
    blk = 1000
    assert n % blk == 0 and blk % 8 == 0
    grid = (n // blk,)
    return pl.pallas_call(
        _tc_body,
        grid=grid,
        in_specs=[
            pl.BlockSpec((blk, d), lambda i: (i, 0)),
            pl.BlockSpec((d, d), lambda i: (0, 0)),
            pl.BlockSpec((d, d), lambda i: (0, 0)),
            pl.BlockSpec((1, d), lambda i: (0, 0)),
        ],
        out_specs=[
            pl.BlockSpec((blk, d), lambda i: (i, 0)),
            pl.BlockSpec((blk, d), lambda i: (i, 0)),
        ],
        out_shape=[
            jax.ShapeDtypeStruct((n, d), jnp.float32),
            jax.ShapeDtypeStruct((n, d), jnp.float32),
        ],
    )(z, W_src, W_dst, bias2d)


@functools.lru_cache(maxsize=None)
def _make_edge_kernel(epad):
    chunks_per_w = epad // (NW * CHUNK)
    edges_per_w = chunks_per_w * CHUNK
    assert chunks_per_w % 2 == 0
    mesh = plsc.VectorSubcoreMesh(core_axis_name="c", subcore_axis_name="s")

    @functools.partial(
        pl.kernel,
        out_type=[
            jax.ShapeDtypeStruct((epad,), jnp.float32),
            jax.ShapeDtypeStruct((epad,), jnp.float32),
        ],
        mesh=mesh,
        compiler_params=pltpu.CompilerParams(needs_layout_passes=False),
        scratch_types=[
            pltpu.VMEM((edges_per_w,), jnp.int32),
            pltpu.VMEM((edges_per_w,), jnp.int32),
            pltpu.VMEM((edges_per_w,), jnp.int32),
            pltpu.VMEM((2, CHUNK, HDIM), jnp.int32),
            pltpu.VMEM((2, CHUNK, HDIM), jnp.int32),
            pltpu.VMEM((2, CHUNK, HDIM), jnp.int32),
            pltpu.VMEM((CHUNK,), jnp.float32),
            pltpu.VMEM((CHUNK,), jnp.float32),
            pltpu.VMEM((DIM + LANES,), jnp.float32),
            pltpu.SemaphoreType.DMA,
            pltpu.SemaphoreType.DMA,
        ],
    )
    def edge_kernel(a_hbm, b_hbm, src_hbm, dst_hbm, neg_hbm, wtab_hbm,
                    pos_hbm, negout_hbm,
                    sidx, didx, nidx, a_rows, b_rows, c_rows,
                    outp, outn, wtab, sem0, sem1):
        wid = lax.axis_index("s") * NC + lax.axis_index("c")
        ebase = wid * edges_per_w
        pltpu.sync_copy(wtab_hbm, wtab)
        pltpu.sync_copy(src_hbm.at[pl.ds(ebase, edges_per_w)], sidx)
        pltpu.sync_copy(dst_hbm.at[pl.ds(ebase, edges_per_w)], didx)
        pltpu.sync_copy(neg_hbm.at[pl.ds(ebase, edges_per_w)], nidx)

        def fire(par, ci, sem):
            off = ci * CHUNK
            pltpu.async_copy(
                a_hbm.at[sidx.at[pl.ds(off, CHUNK)]], a_rows.at[par], sem)
            pltpu.async_copy(
                b_hbm.at[didx.at[pl.ds(off, CHUNK)]], b_rows.at[par], sem)
            pltpu.async_copy(
                b_hbm.at[nidx.at[pl.ds(off, CHUNK)]], c_rows.at[par], sem)

        def drain(par, ci, sem):
            off = ci * CHUNK
            pltpu.make_async_copy(
                a_hbm.at[sidx.at[pl.ds(off, CHUNK)]], a_rows.at[par],
                sem).wait()
            pltpu.make_async_copy(
                b_hbm.at[didx.at[pl.ds(off, CHUNK)]], b_rows.at[par],
                sem).wait()
            pltpu.make_async_copy(
                b_hbm.at[nidx.at[pl.ds(off, CHUNK)]], c_rows.at[par],
                sem).wait()

        bvec = wtab[pl.ds(DIM, LANES)]
        lane = lax.iota(jnp.int32, LANES)

        def unpack(p):
            lo = plsc.bitcast(lax.shift_left(p, 16), jnp.float32)
            hi = plsc.bitcast(
                lax.bitwise_and(p, jnp.int32(-65536)), jnp.float32)
            return lo, hi

        def compute(par, ci):
            def group_body(g, c):
                vecp = jnp.zeros((LANES,), jnp.float32)
                vecn = jnp.zeros((LANES,), jnp.float32)
                for ee in range(LANES):
                    e = g * LANES + ee
                    accp = bvec
                    accn = bvec
                    for k in range(K32):
                        alo, ahi = unpack(a_rows[par, e, pl.ds(k * LANES, LANES)])
                        blo, bhi = unpack(b_rows[par, e, pl.ds(k * LANES, LANES)])
                        clo, chi = unpack(c_rows[par, e, pl.ds(k * LANES, LANES)])
                        wlo = wtab[pl.ds(k * 32, LANES)]
                        whi = wtab[pl.ds(k * 32 + LANES, LANES)]
                        accp = (accp
                                + jnp.maximum(alo + blo, 0.0) * wlo
                                + jnp.maximum(ahi + bhi, 0.0) * whi)
                        accn = (accn
                                + jnp.maximum(alo + clo, 0.0) * wlo
                                + jnp.maximum(ahi + chi, 0.0) * whi)
                    sp = lax.reduce_sum(accp, axes=(0,))
                    sn = lax.reduce_sum(accn, axes=(0,))
                    vecp = jnp.where(lane == ee, sp, vecp)
                    vecn = jnp.where(lane == ee, sn, vecn)
                outp[pl.ds(g * LANES, LANES)] = vecp
                outn[pl.ds(g * LANES, LANES)] = vecn
                return c

            lax.fori_loop(0, CHUNK // LANES, group_body, 0)
            base = ebase + ci * CHUNK
            pltpu.sync_copy(outp, pos_hbm.at[pl.ds(base, CHUNK)])
            pltpu.sync_copy(outn, negout_hbm.at[pl.ds(base, CHUNK)])

        fire(0, 0, sem0)

        def pair_body(i2, c):
            ca = 2 * i2
            fire(1, ca + 1, sem1)
            drain(0, ca, sem0)
            compute(0, ca)

            @pl.when(ca + 2 < chunks_per_w)
            def _():
                fire(0, ca + 2, sem0)

            drain(1, ca + 1, sem1)
            compute(1, ca + 1)
            return c

        lax.fori_loop(0, chunks_per_w // 2, pair_body, 0)

    return edge_kernel


def kernel(z, src_mask, dst_mask, neg_mask, W_src, b_src, W_dst, b_dst,
           W_out, b_out):
    n, d = z.shape
    e = src_mask.shape[0]

    bias2d = (b_src + b_dst).reshape(1, d)
    A, B = _node_transform(z, W_src, W_dst, bias2d)

    A32 = lax.bitcast_convert_type(
        A.astype(jnp.bfloat16).reshape(n, HDIM, 2), jnp.int32)
    B32 = lax.bitcast_convert_type(
        B.astype(jnp.bfloat16).reshape(n, HDIM, 2), jnp.int32)

    w_r = (W_out.reshape(-1).astype(jnp.float32)
           .reshape(K32, LANES, 2).transpose(0, 2, 1).reshape(d))
    wtab = jnp.concatenate(
        [w_r, jnp.pad(b_out.reshape(-1)[:1].astype(jnp.float32),
                      (0, LANES - 1))])

    stride = NW * CHUNK * 2
    epad = ((e + stride - 1) // stride) * stride
    pad = epad - e
    src_p = jnp.concatenate([src_mask.astype(jnp.int32), jnp.zeros((pad,), jnp.int32)])
    dst_p = jnp.concatenate([dst_mask.astype(jnp.int32), jnp.zeros((pad,), jnp.int32)])
    neg_p = jnp.concatenate([neg_mask.astype(jnp.int32), jnp.zeros((pad,), jnp.int32)])

    pos_flat, neg_flat = _make_edge_kernel(epad)(
        A32, B32, src_p, dst_p, neg_p, wtab)

    return (pos_flat[:e].reshape(e, 1), neg_flat[:e].reshape(e, 1))


# speedup vs baseline: 1.0852x; 1.0852x over previous
import functools

import jax
import jax.numpy as jnp
from jax import lax
from jax.experimental import pallas as pl
from jax.experimental.pallas import tpu as pltpu
from jax.experimental.pallas import tpu_sc as plsc

DIM = 256
LANES = 16
K32 = DIM // 32
HDIM = DIM // 2
NC, NS = 2, 16
NW = NC * NS
CHUNK = 128
F0, F1 = 54, 26


def _tc_body(z_ref, ws_ref, wd_ref, bias_ref, a_ref, b_ref):
    zb = z_ref[...]
    a_ref[...] = (
        jnp.dot(zb, ws_ref[...], preferred_element_type=jnp.float32)
        + bias_ref[...]
    )
    b_ref[...] = jnp.dot(zb, wd_ref[...], preferred_element_type=jnp.float32)


def _node_transform(z, W_src, W_dst, bias2d):
    n, d = z.shape
    blk = 1000
    assert n % blk == 0 and blk % 8 == 0
    grid = (n // blk,)
    return pl.pallas_call(
        _tc_body,
        grid=grid,
        in_specs=[
            pl.BlockSpec((blk, d), lambda i: (i, 0)),
            pl.BlockSpec((d, d), lambda i: (0, 0)),
            pl.BlockSpec((d, d), lambda i: (0, 0)),
            pl.BlockSpec((1, d), lambda i: (0, 0)),
        ],
        out_specs=[
            pl.BlockSpec((blk, d), lambda i: (i, 0)),
            pl.BlockSpec((blk, d), lambda i: (i, 0)),
        ],
        out_shape=[
            jax.ShapeDtypeStruct((n, d), jnp.float32),
            jax.ShapeDtypeStruct((n, d), jnp.float32),
        ],
    )(z, W_src, W_dst, bias2d)


@functools.lru_cache(maxsize=None)
def _make_edge_kernel(epad):
    assert epad == NS * (F0 + F1) * CHUNK
    max_edges_w = F0 * CHUNK
    mesh = plsc.VectorSubcoreMesh(core_axis_name="c", subcore_axis_name="s")

    @functools.partial(
        pl.kernel,
        out_type=[
            jax.ShapeDtypeStruct((epad,), jnp.float32),
            jax.ShapeDtypeStruct((epad,), jnp.float32),
        ],
        mesh=mesh,
        compiler_params=pltpu.CompilerParams(needs_layout_passes=False),
        scratch_types=[
            pltpu.VMEM((max_edges_w,), jnp.int32),
            pltpu.VMEM((max_edges_w,), jnp.int32),
            pltpu.VMEM((max_edges_w,), jnp.int32),
            pltpu.VMEM((2, CHUNK, HDIM), jnp.int32),
            pltpu.VMEM((2, CHUNK, HDIM), jnp.int32),
            pltpu.VMEM((2, CHUNK, HDIM), jnp.int32),
            pltpu.VMEM((CHUNK,), jnp.float32),
            pltpu.VMEM((CHUNK,), jnp.float32),
            pltpu.VMEM((DIM + LANES,), jnp.float32),
            pltpu.SemaphoreType.DMA,
            pltpu.SemaphoreType.DMA,
        ],
    )
    def edge_kernel(a_hbm, b_hbm, src_hbm, dst_hbm, neg_hbm, wtab_hbm,
                    pos_hbm, negout_hbm,
                    sidx, didx, nidx, a_rows, b_rows, c_rows,
                    outp, outn, wtab, sem0, sem1):
        c = lax.axis_index("c")
        s = lax.axis_index("s")
        my_chunks = jnp.where(c == 0, F0, F1)
        start_chunk = s * (F0 + F1) + c * F0
        ebase = start_chunk * CHUNK
        pltpu.sync_copy(wtab_hbm, wtab)
        pltpu.sync_copy(src_hbm.at[pl.ds(ebase, max_edges_w)], sidx)
        pltpu.sync_copy(dst_hbm.at[pl.ds(ebase, max_edges_w)], didx)
        pltpu.sync_copy(neg_hbm.at[pl.ds(ebase, max_edges_w)], nidx)

        def fire(par, ci, sem):
            off = ci * CHUNK
            pltpu.async_copy(
                a_hbm.at[sidx.at[pl.ds(off, CHUNK)]], a_rows.at[par], sem)
            pltpu.async_copy(
                b_hbm.at[didx.at[pl.ds(off, CHUNK)]], b_rows.at[par], sem)
            pltpu.async_copy(
                b_hbm.at[nidx.at[pl.ds(off, CHUNK)]], c_rows.at[par], sem)

        def drain(par, ci, sem):
            off = ci * CHUNK
            pltpu.make_async_copy(
                a_hbm.at[sidx.at[pl.ds(off, CHUNK)]], a_rows.at[par],
                sem).wait()
            pltpu.make_async_copy(
                b_hbm.at[didx.at[pl.ds(off, CHUNK)]], b_rows.at[par],
                sem).wait()
            pltpu.make_async_copy(
                b_hbm.at[nidx.at[pl.ds(off, CHUNK)]], c_rows.at[par],
                sem).wait()

        bvec = wtab[pl.ds(DIM, LANES)]
        lane = lax.iota(jnp.int32, LANES)

        def unpack(p):
            lo = plsc.bitcast(lax.shift_left(p, 16), jnp.float32)
            hi = plsc.bitcast(p, jnp.float32)
            return lo, hi

        def compute(par, ci):
            def group_body(g, c):
                vecp = jnp.zeros((LANES,), jnp.float32)
                vecn = jnp.zeros((LANES,), jnp.float32)
                for ee in range(LANES):
                    e = g * LANES + ee
                    accp = bvec
                    accn = bvec
                    for k in range(K32):
                        alo, ahi = unpack(a_rows[par, e, pl.ds(k * LANES, LANES)])
                        blo, bhi = unpack(b_rows[par, e, pl.ds(k * LANES, LANES)])
                        clo, chi = unpack(c_rows[par, e, pl.ds(k * LANES, LANES)])
                        wlo = wtab[pl.ds(k * 32, LANES)]
                        whi = wtab[pl.ds(k * 32 + LANES, LANES)]
                        accp = (accp
                                + jnp.maximum(alo + blo, 0.0) * wlo
                                + jnp.maximum(ahi + bhi, 0.0) * whi)
                        accn = (accn
                                + jnp.maximum(alo + clo, 0.0) * wlo
                                + jnp.maximum(ahi + chi, 0.0) * whi)
                    sp = lax.reduce_sum(accp, axes=(0,))
                    sn = lax.reduce_sum(accn, axes=(0,))
                    vecp = jnp.where(lane == ee, sp, vecp)
                    vecn = jnp.where(lane == ee, sn, vecn)
                outp[pl.ds(g * LANES, LANES)] = vecp
                outn[pl.ds(g * LANES, LANES)] = vecn
                return c

            lax.fori_loop(0, CHUNK // LANES, group_body, 0)
            base = ebase + ci * CHUNK
            pltpu.sync_copy(outp, pos_hbm.at[pl.ds(base, CHUNK)])
            pltpu.sync_copy(outn, negout_hbm.at[pl.ds(base, CHUNK)])

        fire(0, 0, sem0)

        def pair_body(i2, carry):
            ca = 2 * i2
            fire(1, ca + 1, sem1)
            drain(0, ca, sem0)
            compute(0, ca)

            @pl.when(ca + 2 < my_chunks)
            def _():
                fire(0, ca + 2, sem0)

            drain(1, ca + 1, sem1)
            compute(1, ca + 1)
            return carry

        lax.fori_loop(0, my_chunks // 2, pair_body, 0)

    return edge_kernel


def kernel(z, src_mask, dst_mask, neg_mask, W_src, b_src, W_dst, b_dst,
           W_out, b_out):
    n, d = z.shape
    e = src_mask.shape[0]

    bias2d = (b_src + b_dst).reshape(1, d)
    A, B = _node_transform(z, W_src, W_dst, bias2d)

    A32 = lax.bitcast_convert_type(
        A.astype(jnp.bfloat16).reshape(n, HDIM, 2), jnp.int32)
    B32 = lax.bitcast_convert_type(
        B.astype(jnp.bfloat16).reshape(n, HDIM, 2), jnp.int32)

    w_r = (W_out.reshape(-1).astype(jnp.float32)
           .reshape(K32, LANES, 2).transpose(0, 2, 1).reshape(d))
    wtab = jnp.concatenate(
        [w_r, jnp.pad(b_out.reshape(-1)[:1].astype(jnp.float32),
                      (0, LANES - 1))])

    stride = NS * (F0 + F1) * CHUNK
    epad = ((e + stride - 1) // stride) * stride
    pad = epad - e + (F0 - F1) * CHUNK
    src_p = jnp.concatenate([src_mask.astype(jnp.int32), jnp.zeros((pad,), jnp.int32)])
    dst_p = jnp.concatenate([dst_mask.astype(jnp.int32), jnp.zeros((pad,), jnp.int32)])
    neg_p = jnp.concatenate([neg_mask.astype(jnp.int32), jnp.zeros((pad,), jnp.int32)])

    pos_flat, neg_flat = _make_edge_kernel(epad)(
        A32, B32, src_p, dst_p, neg_p, wtab)

    return (pos_flat[:e].reshape(e, 1), neg_flat[:e].reshape(e, 1))


# speedup vs baseline: 1.4919x; 1.3747x over previous
import functools

import jax
import jax.numpy as jnp
from jax import lax
from jax.experimental import pallas as pl
from jax.experimental.pallas import tpu as pltpu
from jax.experimental.pallas import tpu_sc as plsc

DIM = 256
LANES = 16
K32 = DIM // 32
HDIM = DIM // 2
NC, NS = 2, 16
NW = NC * NS
CHUNK = 128
F0, F1 = 54, 26


def _pack_bf16_pairs(x):
    h = x.shape[-1] // 2
    xb = lax.bitcast_convert_type(x, jnp.int32)
    r = xb + 0x7FFF + lax.bitwise_and(lax.shift_right_logical(xb, 16), 1)
    lo = lax.shift_right_logical(r[:, :h], 16)
    hi = lax.bitwise_and(r[:, h:], jnp.int32(-65536))
    return lax.bitwise_or(hi, lo)


def _tc_body(z_ref, ws_ref, wd_ref, bias_ref, a_ref, b_ref):
    zb = z_ref[...]
    af = (jnp.dot(zb, ws_ref[...], preferred_element_type=jnp.float32)
          + bias_ref[...])
    bf = jnp.dot(zb, wd_ref[...], preferred_element_type=jnp.float32)
    a_ref[...] = _pack_bf16_pairs(af)
    b_ref[...] = _pack_bf16_pairs(bf)


def _node_transform(z, W_src, W_dst, bias2d):
    n, d = z.shape
    blk = 1000
    assert n % blk == 0 and blk % 8 == 0
    grid = (n // blk,)
    return pl.pallas_call(
        _tc_body,
        grid=grid,
        in_specs=[
            pl.BlockSpec((blk, d), lambda i: (i, 0)),
            pl.BlockSpec((d, d), lambda i: (0, 0)),
            pl.BlockSpec((d, d), lambda i: (0, 0)),
            pl.BlockSpec((1, d), lambda i: (0, 0)),
        ],
        out_specs=[
            pl.BlockSpec((blk, d // 2), lambda i: (i, 0)),
            pl.BlockSpec((blk, d // 2), lambda i: (i, 0)),
        ],
        out_shape=[
            jax.ShapeDtypeStruct((n, d // 2), jnp.int32),
            jax.ShapeDtypeStruct((n, d // 2), jnp.int32),
        ],
    )(z, W_src, W_dst, bias2d)


@functools.lru_cache(maxsize=None)
def _make_edge_kernel(epad):
    assert epad == NS * (F0 + F1) * CHUNK
    max_edges_w = F0 * CHUNK
    mesh = plsc.VectorSubcoreMesh(core_axis_name="c", subcore_axis_name="s")

    @functools.partial(
        pl.kernel,
        out_type=[
            jax.ShapeDtypeStruct((epad,), jnp.float32),
            jax.ShapeDtypeStruct((epad,), jnp.float32),
        ],
        mesh=mesh,
        compiler_params=pltpu.CompilerParams(needs_layout_passes=False),
        scratch_types=[
            pltpu.VMEM((max_edges_w,), jnp.int32),
            pltpu.VMEM((max_edges_w,), jnp.int32),
            pltpu.VMEM((max_edges_w,), jnp.int32),
            pltpu.VMEM((2, CHUNK, HDIM), jnp.int32),
            pltpu.VMEM((2, CHUNK, HDIM), jnp.int32),
            pltpu.VMEM((2, CHUNK, HDIM), jnp.int32),
            pltpu.VMEM((CHUNK,), jnp.float32),
            pltpu.VMEM((CHUNK,), jnp.float32),
            pltpu.VMEM((DIM + LANES,), jnp.float32),
            pltpu.SemaphoreType.DMA,
            pltpu.SemaphoreType.DMA,
        ],
    )
    def edge_kernel(a_hbm, b_hbm, src_hbm, dst_hbm, neg_hbm, wtab_hbm,
                    pos_hbm, negout_hbm,
                    sidx, didx, nidx, a_rows, b_rows, c_rows,
                    outp, outn, wtab, sem0, sem1):
        c = lax.axis_index("c")
        s = lax.axis_index("s")
        my_chunks = jnp.where(c == 0, F0, F1)
        start_chunk = s * (F0 + F1) + c * F0
        ebase = start_chunk * CHUNK
        cp_w = pltpu.async_copy(wtab_hbm, wtab, sem0)
        cp_s = pltpu.async_copy(src_hbm.at[pl.ds(ebase, max_edges_w)], sidx, sem0)
        cp_d = pltpu.async_copy(dst_hbm.at[pl.ds(ebase, max_edges_w)], didx, sem0)
        cp_n = pltpu.async_copy(neg_hbm.at[pl.ds(ebase, max_edges_w)], nidx, sem0)
        cp_w.wait()
        cp_s.wait()
        cp_d.wait()
        cp_n.wait()

        def fire(par, ci, sem):
            off = ci * CHUNK
            pltpu.async_copy(
                a_hbm.at[sidx.at[pl.ds(off, CHUNK)]], a_rows.at[par], sem)
            pltpu.async_copy(
                b_hbm.at[didx.at[pl.ds(off, CHUNK)]], b_rows.at[par], sem)
            pltpu.async_copy(
                b_hbm.at[nidx.at[pl.ds(off, CHUNK)]], c_rows.at[par], sem)

        def drain(par, ci, sem):
            off = ci * CHUNK
            pltpu.make_async_copy(
                a_hbm.at[sidx.at[pl.ds(off, CHUNK)]], a_rows.at[par],
                sem).wait()
            pltpu.make_async_copy(
                b_hbm.at[didx.at[pl.ds(off, CHUNK)]], b_rows.at[par],
                sem).wait()
            pltpu.make_async_copy(
                b_hbm.at[nidx.at[pl.ds(off, CHUNK)]], c_rows.at[par],
                sem).wait()

        bvec = wtab[pl.ds(DIM, LANES)]
        lane = lax.iota(jnp.int32, LANES)

        def unpack(p):
            lo = plsc.bitcast(lax.shift_left(p, 16), jnp.float32)
            hi = plsc.bitcast(p, jnp.float32)
            return lo, hi

        def compute(par, ci):
            def group_body(g, c):
                vecp = jnp.zeros((LANES,), jnp.float32)
                vecn = jnp.zeros((LANES,), jnp.float32)
                for ee in range(LANES):
                    e = g * LANES + ee
                    accp = bvec
                    accn = bvec
                    for k in range(K32):
                        alo, ahi = unpack(a_rows[par, e, pl.ds(k * LANES, LANES)])
                        blo, bhi = unpack(b_rows[par, e, pl.ds(k * LANES, LANES)])
                        clo, chi = unpack(c_rows[par, e, pl.ds(k * LANES, LANES)])
                        wlo = wtab[pl.ds(k * 32, LANES)]
                        whi = wtab[pl.ds(k * 32 + LANES, LANES)]
                        accp = (accp
                                + jnp.maximum(alo + blo, 0.0) * wlo
                                + jnp.maximum(ahi + bhi, 0.0) * whi)
                        accn = (accn
                                + jnp.maximum(alo + clo, 0.0) * wlo
                                + jnp.maximum(ahi + chi, 0.0) * whi)
                    sp = lax.reduce_sum(accp, axes=(0,))
                    sn = lax.reduce_sum(accn, axes=(0,))
                    vecp = jnp.where(lane == ee, sp, vecp)
                    vecn = jnp.where(lane == ee, sn, vecn)
                outp[pl.ds(g * LANES, LANES)] = vecp
                outn[pl.ds(g * LANES, LANES)] = vecn
                return c

            lax.fori_loop(0, CHUNK // LANES, group_body, 0)
            base = ebase + ci * CHUNK
            pltpu.sync_copy(outp, pos_hbm.at[pl.ds(base, CHUNK)])
            pltpu.sync_copy(outn, negout_hbm.at[pl.ds(base, CHUNK)])

        fire(0, 0, sem0)

        def pair_body(i2, carry):
            ca = 2 * i2
            fire(1, ca + 1, sem1)
            drain(0, ca, sem0)
            compute(0, ca)

            @pl.when(ca + 2 < my_chunks)
            def _():
                fire(0, ca + 2, sem0)

            drain(1, ca + 1, sem1)
            compute(1, ca + 1)
            return carry

        lax.fori_loop(0, my_chunks // 2, pair_body, 0)

    return edge_kernel


def kernel(z, src_mask, dst_mask, neg_mask, W_src, b_src, W_dst, b_dst,
           W_out, b_out):
    n, d = z.shape
    e = src_mask.shape[0]

    bias2d = (b_src + b_dst).reshape(1, d)
    A32, B32 = _node_transform(z, W_src, W_dst, bias2d)

    w_r = (W_out.reshape(-1).astype(jnp.float32)
           .reshape(2, K32, LANES).transpose(1, 0, 2).reshape(d))
    wtab = jnp.concatenate(
        [w_r, jnp.pad(b_out.reshape(-1)[:1].astype(jnp.float32),
                      (0, LANES - 1))])

    stride = NS * (F0 + F1) * CHUNK
    epad = ((e + stride - 1) // stride) * stride
    pad = epad - e + (F0 - F1) * CHUNK
    src_p = jnp.concatenate([src_mask.astype(jnp.int32), jnp.zeros((pad,), jnp.int32)])
    dst_p = jnp.concatenate([dst_mask.astype(jnp.int32), jnp.zeros((pad,), jnp.int32)])
    neg_p = jnp.concatenate([neg_mask.astype(jnp.int32), jnp.zeros((pad,), jnp.int32)])

    pos_flat, neg_flat = _make_edge_kernel(epad)(
        A32, B32, src_p, dst_p, neg_p, wtab)

    return (pos_flat[:e].reshape(e, 1), neg_flat[:e].reshape(e, 1))


# speedup vs baseline: 1.5106x; 1.0125x over previous
import functools

import jax
import jax.numpy as jnp
from jax import lax
from jax.experimental import pallas as pl
from jax.experimental.pallas import tpu as pltpu
from jax.experimental.pallas import tpu_sc as plsc

DIM = 256
LANES = 16
K32 = DIM // 32
HDIM = DIM // 2
NC, NS = 2, 16
NW = NC * NS
CHUNK = 128
F0, F1 = 54, 26


def _pack_bf16_pairs(x):
    h = x.shape[-1] // 2
    xb = lax.bitcast_convert_type(x, jnp.int32)
    r = xb + 0x7FFF + lax.bitwise_and(lax.shift_right_logical(xb, 16), 1)
    lo = lax.shift_right_logical(r[:, :h], 16)
    hi = lax.bitwise_and(r[:, h:], jnp.int32(-65536))
    return lax.bitwise_or(hi, lo)


def _tc_body(z_ref, ws_ref, wd_ref, bias_ref, a_ref, b_ref):
    zb = z_ref[...]
    af = (jnp.dot(zb, ws_ref[...], preferred_element_type=jnp.float32)
          + bias_ref[...])
    bf = jnp.dot(zb, wd_ref[...], preferred_element_type=jnp.float32)
    a_ref[...] = _pack_bf16_pairs(af)
    b_ref[...] = _pack_bf16_pairs(bf)


def _node_transform(z, W_src, W_dst, bias2d):
    n, d = z.shape
    blk = 1000
    assert n % blk == 0 and blk % 8 == 0
    grid = (n // blk,)
    return pl.pallas_call(
        _tc_body,
        grid=grid,
        in_specs=[
            pl.BlockSpec((blk, d), lambda i: (i, 0)),
            pl.BlockSpec((d, d), lambda i: (0, 0)),
            pl.BlockSpec((d, d), lambda i: (0, 0)),
            pl.BlockSpec((1, d), lambda i: (0, 0)),
        ],
        out_specs=[
            pl.BlockSpec((blk, d // 2), lambda i: (i, 0)),
            pl.BlockSpec((blk, d // 2), lambda i: (i, 0)),
        ],
        out_shape=[
            jax.ShapeDtypeStruct((n, d // 2), jnp.int32),
            jax.ShapeDtypeStruct((n, d // 2), jnp.int32),
        ],
    )(z, W_src, W_dst, bias2d)


@functools.lru_cache(maxsize=None)
def _make_edge_kernel(epad):
    assert epad == NS * (F0 + F1) * CHUNK
    max_edges_w = F0 * CHUNK
    mesh = plsc.VectorSubcoreMesh(core_axis_name="c", subcore_axis_name="s")

    @functools.partial(
        pl.kernel,
        out_type=[
            jax.ShapeDtypeStruct((epad,), jnp.float32),
            jax.ShapeDtypeStruct((epad,), jnp.float32),
        ],
        mesh=mesh,
        compiler_params=pltpu.CompilerParams(needs_layout_passes=False),
        scratch_types=[
            pltpu.VMEM((max_edges_w,), jnp.int32),
            pltpu.VMEM((max_edges_w,), jnp.int32),
            pltpu.VMEM((max_edges_w,), jnp.int32),
            pltpu.VMEM((2, CHUNK, HDIM), jnp.int32),
            pltpu.VMEM((2, CHUNK, HDIM), jnp.int32),
            pltpu.VMEM((2, CHUNK, HDIM), jnp.int32),
            pltpu.VMEM((CHUNK,), jnp.float32),
            pltpu.VMEM((CHUNK,), jnp.float32),
            pltpu.VMEM((DIM + LANES,), jnp.float32),
            pltpu.SemaphoreType.DMA,
            pltpu.SemaphoreType.DMA,
        ],
    )
    def edge_kernel(a_hbm, b_hbm, src_hbm, dst_hbm, neg_hbm, wtab_hbm,
                    pos_hbm, negout_hbm,
                    sidx, didx, nidx, a_rows, b_rows, c_rows,
                    outp, outn, wtab, sem0, sem1):
        c = lax.axis_index("c")
        s = lax.axis_index("s")
        my_chunks = jnp.where(c == 0, F0, F1)
        start_chunk = s * (F0 + F1) + c * F0
        ebase = start_chunk * CHUNK
        cp_w = pltpu.async_copy(wtab_hbm, wtab, sem0)
        cp_s = pltpu.async_copy(src_hbm.at[pl.ds(ebase, max_edges_w)], sidx, sem0)
        cp_d = pltpu.async_copy(dst_hbm.at[pl.ds(ebase, max_edges_w)], didx, sem0)
        cp_n = pltpu.async_copy(neg_hbm.at[pl.ds(ebase, max_edges_w)], nidx, sem0)
        cp_w.wait()
        cp_s.wait()
        cp_d.wait()
        cp_n.wait()

        def fire(par, ci, sem):
            off = ci * CHUNK
            pltpu.async_copy(
                a_hbm.at[sidx.at[pl.ds(off, CHUNK)]], a_rows.at[par], sem)
            pltpu.async_copy(
                b_hbm.at[didx.at[pl.ds(off, CHUNK)]], b_rows.at[par], sem)
            pltpu.async_copy(
                b_hbm.at[nidx.at[pl.ds(off, CHUNK)]], c_rows.at[par], sem)

        def drain(par, ci, sem):
            off = ci * CHUNK
            pltpu.make_async_copy(
                a_hbm.at[sidx.at[pl.ds(off, CHUNK)]], a_rows.at[par],
                sem).wait()
            pltpu.make_async_copy(
                b_hbm.at[didx.at[pl.ds(off, CHUNK)]], b_rows.at[par],
                sem).wait()
            pltpu.make_async_copy(
                b_hbm.at[nidx.at[pl.ds(off, CHUNK)]], c_rows.at[par],
                sem).wait()

        bvec = wtab[pl.ds(DIM, LANES)]
        lane = lax.iota(jnp.int32, LANES)

        def unpack(p):
            lo = plsc.bitcast(lax.shift_left(p, 16), jnp.float32)
            hi = plsc.bitcast(p, jnp.float32)
            return lo, hi

        def compute(par, ci):
            def group_body(g, c):
                def edge_body(ee, vec):
                    vecp, vecn = vec
                    e = g * LANES + ee
                    accp = bvec
                    accn = bvec
                    for k in range(K32):
                        alo, ahi = unpack(a_rows[par, e, pl.ds(k * LANES, LANES)])
                        blo, bhi = unpack(b_rows[par, e, pl.ds(k * LANES, LANES)])
                        clo, chi = unpack(c_rows[par, e, pl.ds(k * LANES, LANES)])
                        wlo = wtab[pl.ds(k * 32, LANES)]
                        whi = wtab[pl.ds(k * 32 + LANES, LANES)]
                        accp = (accp
                                + jnp.maximum(alo + blo, 0.0) * wlo
                                + jnp.maximum(ahi + bhi, 0.0) * whi)
                        accn = (accn
                                + jnp.maximum(alo + clo, 0.0) * wlo
                                + jnp.maximum(ahi + chi, 0.0) * whi)
                    sp = lax.reduce_sum(accp, axes=(0,))
                    sn = lax.reduce_sum(accn, axes=(0,))
                    m = lane == ee
                    return (jnp.where(m, sp, vecp), jnp.where(m, sn, vecn))

                vecp, vecn = lax.fori_loop(
                    0, LANES, edge_body,
                    (jnp.zeros((LANES,), jnp.float32),
                     jnp.zeros((LANES,), jnp.float32)))
                outp[pl.ds(g * LANES, LANES)] = vecp
                outn[pl.ds(g * LANES, LANES)] = vecn
                return c

            lax.fori_loop(0, CHUNK // LANES, group_body, 0)
            base = ebase + ci * CHUNK
            pltpu.sync_copy(outp, pos_hbm.at[pl.ds(base, CHUNK)])
            pltpu.sync_copy(outn, negout_hbm.at[pl.ds(base, CHUNK)])

        fire(0, 0, sem0)

        def pair_body(i2, carry):
            ca = 2 * i2
            fire(1, ca + 1, sem1)
            drain(0, ca, sem0)
            compute(0, ca)

            @pl.when(ca + 2 < my_chunks)
            def _():
                fire(0, ca + 2, sem0)

            drain(1, ca + 1, sem1)
            compute(1, ca + 1)
            return carry

        lax.fori_loop(0, my_chunks // 2, pair_body, 0)

    return edge_kernel


def kernel(z, src_mask, dst_mask, neg_mask, W_src, b_src, W_dst, b_dst,
           W_out, b_out):
    n, d = z.shape
    e = src_mask.shape[0]

    bias2d = (b_src + b_dst).reshape(1, d)
    A32, B32 = _node_transform(z, W_src, W_dst, bias2d)

    w_r = (W_out.reshape(-1).astype(jnp.float32)
           .reshape(2, K32, LANES).transpose(1, 0, 2).reshape(d))
    wtab = jnp.concatenate(
        [w_r, jnp.pad(b_out.reshape(-1)[:1].astype(jnp.float32),
                      (0, LANES - 1))])

    stride = NS * (F0 + F1) * CHUNK
    epad = ((e + stride - 1) // stride) * stride
    pad = epad - e + (F0 - F1) * CHUNK
    src_p = jnp.concatenate([src_mask.astype(jnp.int32), jnp.zeros((pad,), jnp.int32)])
    dst_p = jnp.concatenate([dst_mask.astype(jnp.int32), jnp.zeros((pad,), jnp.int32)])
    neg_p = jnp.concatenate([neg_mask.astype(jnp.int32), jnp.zeros((pad,), jnp.int32)])

    pos_flat, neg_flat = _make_edge_kernel(epad)(
        A32, B32, src_p, dst_p, neg_p, wtab)

    return (pos_flat[:e].reshape(e, 1), neg_flat[:e].reshape(e, 1))
